# initial kernel scaffold (unmeasured)
import jax
import jax.numpy as jnp
from jax import lax
from jax.experimental import pallas as pl
from jax.experimental.pallas import tpu as pltpu

N_DEV = 4


def kernel(A, B):
    m, _ = A.shape
    _, n = B.shape
    ch = m // N_DEV

    A = A.astype(jnp.bfloat16)
    B = B.astype(jnp.bfloat16)

    def body(a_ref, b_ref, out_ref, send_ref, recv_ref, send_sems, recv_sems):
        my = lax.axis_index("i")

        rdmas = []
        for s in range(N_DEV - 1):
            tgt = (my + 1 + s) % N_DEV
            part = jnp.dot(
                a_ref[pl.ds(tgt * ch, ch), :],
                b_ref[:, :],
                preferred_element_type=jnp.float32,
            )
            send_ref[s] = part.astype(jnp.bfloat16)
            rdma = pltpu.make_async_remote_copy(
                src_ref=send_ref.at[s],
                dst_ref=recv_ref.at[s],
                send_sem=send_sems.at[s],
                recv_sem=recv_sems.at[s],
                device_id=(tgt,),
                device_id_type=pl.DeviceIdType.MESH,
            )
            rdma.start()
            rdmas.append(rdma)

        acc = jnp.dot(
            a_ref[pl.ds(my * ch, ch), :],
            b_ref[:, :],
            preferred_element_type=jnp.float32,
        )

        for s in range(N_DEV - 1):
            rdmas[s].wait_recv()
            acc = acc + recv_ref[s].astype(jnp.float32)
        out_ref[:, :] = acc

        for s in range(N_DEV - 1):
            rdmas[s].wait_send()

    return pl.pallas_call(
        body,
        out_shape=jax.ShapeDtypeStruct((ch, n), jnp.float32),
        in_specs=[
            pl.BlockSpec(memory_space=pltpu.VMEM),
            pl.BlockSpec(memory_space=pltpu.VMEM),
        ],
        out_specs=pl.BlockSpec(memory_space=pltpu.VMEM),
        scratch_shapes=[
            pltpu.VMEM((N_DEV - 1, ch, n), jnp.bfloat16),
            pltpu.VMEM((N_DEV - 1, ch, n), jnp.bfloat16),
            pltpu.SemaphoreType.DMA((N_DEV - 1,)),
            pltpu.SemaphoreType.DMA((N_DEV - 1,)),
        ],
        compiler_params=pltpu.CompilerParams(collective_id=0),
    )(A, B)


# baseline (device time: 158500 ns/iter reference)
import jax
import jax.numpy as jnp
from jax import lax
from jax.experimental import pallas as pl
from jax.experimental.pallas import tpu as pltpu

N_DEV = 4


def kernel(A, B):
    m, _ = A.shape
    _, n = B.shape
    ch = m // N_DEV

    A = A.astype(jnp.bfloat16)
    B = B.astype(jnp.bfloat16)

    def body(a_ref, b_ref, out_ref, send_ref, recv_ref, send_sems, recv_sems):
        my = lax.axis_index("i")

        rdmas = []
        for s in range(N_DEV - 1):
            tgt = (my + 1 + s) % N_DEV
            send_ref[s] = jnp.dot(
                a_ref[pl.ds(tgt * ch, ch), :],
                b_ref[:, :],
                preferred_element_type=jnp.float32,
            ).astype(jnp.bfloat16)
            rdma = pltpu.make_async_remote_copy(
                src_ref=send_ref.at[s],
                dst_ref=recv_ref.at[s],
                send_sem=send_sems.at[s],
                recv_sem=recv_sems.at[s],
                device_id=(tgt,),
                device_id_type=pl.DeviceIdType.MESH,
            )
            rdma.start()
            rdmas.append(rdma)

        out_ref[:, :] = jnp.dot(
            a_ref[pl.ds(my * ch, ch), :],
            b_ref[:, :],
            preferred_element_type=jnp.float32,
        ).astype(jnp.bfloat16)

        for s in range(N_DEV - 1):
            rdmas[s].wait_recv()
            out_ref[:, :] = out_ref[:, :] + recv_ref[s]

        for s in range(N_DEV - 1):
            rdmas[s].wait_send()

    return pl.pallas_call(
        body,
        out_shape=jax.ShapeDtypeStruct((ch, n), jnp.bfloat16),
        in_specs=[
            pl.BlockSpec(memory_space=pltpu.VMEM),
            pl.BlockSpec(memory_space=pltpu.VMEM),
        ],
        out_specs=pl.BlockSpec(memory_space=pltpu.VMEM),
        scratch_shapes=[
            pltpu.VMEM((N_DEV - 1, ch, n), jnp.bfloat16),
            pltpu.VMEM((N_DEV - 1, ch, n), jnp.bfloat16),
            pltpu.SemaphoreType.DMA((N_DEV - 1,)),
            pltpu.SemaphoreType.DMA((N_DEV - 1,)),
        ],
        compiler_params=pltpu.CompilerParams(
            vmem_limit_bytes=110 * 1024 * 1024,
        ),
    )(A, B)


# device time: 124537 ns/iter; 1.2727x vs baseline; 1.2727x over previous
import jax
import jax.numpy as jnp
from jax import lax
from jax.experimental import pallas as pl
from jax.experimental.pallas import tpu as pltpu

N_DEV = 4


def kernel(A, B):
    m, _ = A.shape
    _, n = B.shape
    ch = m // N_DEV
    h = n // 2

    A = A.astype(jnp.bfloat16)
    B = B.astype(jnp.bfloat16)

    def body(a_ref, b_ref, out_ref, send_ref, recv_ref, send_sems, recv_sems):
        my = lax.axis_index("i")
        right = (my + 1) % N_DEV
        left = (my + 3) % N_DEV
        diag = (my + 2) % N_DEV

        def mm(rows, cols):
            return jnp.dot(
                a_ref[pl.ds(rows * ch, ch), :],
                b_ref[:, cols],
                preferred_element_type=jnp.float32,
            ).astype(jnp.bfloat16)

        def make(slot, tgt):
            return pltpu.make_async_remote_copy(
                src_ref=send_ref.at[slot],
                dst_ref=recv_ref.at[slot],
                send_sem=send_sems.at[slot],
                recv_sem=recv_sems.at[slot],
                device_id=(tgt,),
                device_id_type=pl.DeviceIdType.MESH,
            )

        L = slice(0, h)
        R = slice(h, n)
        tgts = [right, right, right, left, left, left]
        rd = [make(s, tgts[s]) for s in range(6)]

        send_ref[0] = mm(diag, L)
        rd[0].start()
        send_ref[3] = mm(diag, R)
        rd[3].start()

        send_ref[1] = mm(right, R)
        rd[1].start()
        send_ref[4] = mm(left, L)
        rd[4].start()

        send_ref[2] = mm(right, L)
        send_ref[5] = mm(left, R)

        rd[0].wait_recv()
        send_ref[2] = send_ref[2] + recv_ref[0]
        rd[2].start()
        rd[3].wait_recv()
        send_ref[5] = send_ref[5] + recv_ref[3]
        rd[5].start()

        out_ref[:, L] = mm(my, L)
        out_ref[:, R] = mm(my, R)

        rd[2].wait_recv()
        rd[4].wait_recv()
        out_ref[:, L] = out_ref[:, L] + (recv_ref[2] + recv_ref[4])
        rd[5].wait_recv()
        rd[1].wait_recv()
        out_ref[:, R] = out_ref[:, R] + (recv_ref[5] + recv_ref[1])

        for s in range(6):
            rd[s].wait_send()

    return pl.pallas_call(
        body,
        out_shape=jax.ShapeDtypeStruct((ch, n), jnp.bfloat16),
        in_specs=[
            pl.BlockSpec(memory_space=pltpu.VMEM),
            pl.BlockSpec(memory_space=pltpu.VMEM),
        ],
        out_specs=pl.BlockSpec(memory_space=pltpu.VMEM),
        scratch_shapes=[
            pltpu.VMEM((6, ch, h), jnp.bfloat16),
            pltpu.VMEM((6, ch, h), jnp.bfloat16),
            pltpu.SemaphoreType.DMA((6,)),
            pltpu.SemaphoreType.DMA((6,)),
        ],
        compiler_params=pltpu.CompilerParams(
            vmem_limit_bytes=110 * 1024 * 1024,
        ),
    )(A, B)


# device time: 116481 ns/iter; 1.3607x vs baseline; 1.0692x over previous
import jax
import jax.numpy as jnp
from jax import lax
from jax.experimental import pallas as pl
from jax.experimental.pallas import tpu as pltpu

N_DEV = 4


def kernel(A, B):
    m, _ = A.shape
    _, n = B.shape
    ch = m // N_DEV
    h = n // 2

    B = B.astype(jnp.bfloat16)

    def body(
        a_hbm,
        b_ref,
        out_ref,
        send_ref,
        recv_ref,
        stage_ref,
        a16_ref,
        send_sems,
        recv_sems,
        dma_sems,
    ):
        my = lax.axis_index("i")
        right = (my + 1) % N_DEV
        left = (my + 3) % N_DEV
        diag = (my + 2) % N_DEV

        def fetch(slot, rows):
            return pltpu.make_async_copy(
                a_hbm.at[pl.ds(rows * ch, ch), :],
                stage_ref.at[slot],
                dma_sems.at[slot],
            )

        def mm(slot, cols):
            return jnp.dot(
                a16_ref[slot, :, :],
                b_ref[:, cols],
                preferred_element_type=jnp.float32,
            ).astype(jnp.bfloat16)

        def make(slot, tgt):
            return pltpu.make_async_remote_copy(
                src_ref=send_ref.at[slot],
                dst_ref=recv_ref.at[slot],
                send_sem=send_sems.at[slot],
                recv_sem=recv_sems.at[slot],
                device_id=(tgt,),
                device_id_type=pl.DeviceIdType.MESH,
            )

        L = slice(0, h)
        R = slice(h, n)
        tgts = [right, right, right, left, left, left]
        rd = [make(s, tgts[s]) for s in range(6)]

        f0 = fetch(0, diag)
        f0.start()
        f1 = fetch(1, right)
        f1.start()

        f0.wait()
        a16_ref[0] = stage_ref[0].astype(jnp.bfloat16)

        send_ref[0] = mm(0, L)
        rd[0].start()
        send_ref[3] = mm(0, R)
        rd[3].start()

        f1.wait()
        a16_ref[1] = stage_ref[1].astype(jnp.bfloat16)
        f0b = fetch(0, left)
        f0b.start()

        send_ref[1] = mm(1, R)
        rd[1].start()
        send_ref[2] = mm(1, L)

        f0b.wait()
        a16_ref[0] = stage_ref[0].astype(jnp.bfloat16)
        f1b = fetch(1, my)
        f1b.start()

        send_ref[4] = mm(0, L)
        rd[4].start()
        send_ref[5] = mm(0, R)

        rd[0].wait_recv()
        send_ref[2] = send_ref[2] + recv_ref[0]
        rd[2].start()
        rd[3].wait_recv()
        send_ref[5] = send_ref[5] + recv_ref[3]
        rd[5].start()

        f1b.wait()
        a16_ref[1] = stage_ref[1].astype(jnp.bfloat16)
        out_ref[:, L] = mm(1, L)
        out_ref[:, R] = mm(1, R)

        rd[4].wait_recv()
        rd[2].wait_recv()
        out_ref[:, L] = out_ref[:, L] + (recv_ref[2] + recv_ref[4])
        rd[1].wait_recv()
        rd[5].wait_recv()
        out_ref[:, R] = out_ref[:, R] + (recv_ref[5] + recv_ref[1])

        for s in range(6):
            rd[s].wait_send()

    k = A.shape[1]
    return pl.pallas_call(
        body,
        out_shape=jax.ShapeDtypeStruct((ch, n), jnp.bfloat16),
        in_specs=[
            pl.BlockSpec(memory_space=pltpu.MemorySpace.HBM),
            pl.BlockSpec(memory_space=pltpu.VMEM),
        ],
        out_specs=pl.BlockSpec(memory_space=pltpu.VMEM),
        scratch_shapes=[
            pltpu.VMEM((6, ch, h), jnp.bfloat16),
            pltpu.VMEM((6, ch, h), jnp.bfloat16),
            pltpu.VMEM((2, ch, k), jnp.float32),
            pltpu.VMEM((2, ch, k), jnp.bfloat16),
            pltpu.SemaphoreType.DMA((6,)),
            pltpu.SemaphoreType.DMA((6,)),
            pltpu.SemaphoreType.DMA((2,)),
        ],
        compiler_params=pltpu.CompilerParams(
            vmem_limit_bytes=110 * 1024 * 1024,
        ),
    )(A, B)


# device time: 111721 ns/iter; 1.4187x vs baseline; 1.0426x over previous
import jax
import jax.numpy as jnp
from jax import lax
from jax.experimental import pallas as pl
from jax.experimental.pallas import tpu as pltpu

N_DEV = 4


def kernel(A, B):
    m, k = A.shape
    _, n = B.shape
    ch = m // N_DEV
    h = n // 2
    q = n // 4

    def body(
        a_hbm,
        b_hbm,
        out_ref,
        send_ref,
        recv_ref,
        astage_ref,
        a16_ref,
        bstage_ref,
        b16_ref,
        send_sems,
        recv_sems,
        adma_sem,
        bdma_sem,
    ):
        my = lax.axis_index("i")
        right = (my + 1) % N_DEV
        left = (my + 3) % N_DEV
        diag = (my + 2) % N_DEV

        def fetch_a(rows):
            return pltpu.make_async_copy(
                a_hbm.at[pl.ds(rows * ch, ch), :], astage_ref, adma_sem
            )

        def fetch_b(j):
            return pltpu.make_async_copy(
                b_hbm.at[:, j * q:(j + 1) * q], bstage_ref, bdma_sem
            )

        def mm(slot, cols):
            return jnp.dot(
                a16_ref[slot, :, :],
                b16_ref[:, cols],
                preferred_element_type=jnp.float32,
            ).astype(jnp.bfloat16)

        def make(slot, tgt):
            return pltpu.make_async_remote_copy(
                src_ref=send_ref.at[slot],
                dst_ref=recv_ref.at[slot],
                send_sem=send_sems.at[slot],
                recv_sem=recv_sems.at[slot],
                device_id=(tgt,),
                device_id_type=pl.DeviceIdType.MESH,
            )

        L = slice(0, h)
        R = slice(h, n)
        tgts = [right, right, right, left, left, left]
        rd = [make(s, tgts[s]) for s in range(6)]

        fa = fetch_a(diag)
        fa.start()
        fb = fetch_b(0)
        fb.start()
        fa.wait()
        a16_ref[0] = astage_ref[:, :].astype(jnp.bfloat16)
        fa = fetch_a(right)
        fa.start()
        fb.wait()
        b16_ref[:, 0:q] = bstage_ref[:, :].astype(jnp.bfloat16)
        fb = fetch_b(1)
        fb.start()
        fb.wait()
        b16_ref[:, q:h] = bstage_ref[:, :].astype(jnp.bfloat16)
        fb = fetch_b(2)
        fb.start()

        send_ref[0] = mm(0, L)
        rd[0].start()

        fb.wait()
        b16_ref[:, h:h + q] = bstage_ref[:, :].astype(jnp.bfloat16)
        fb = fetch_b(3)
        fb.start()
        fb.wait()
        b16_ref[:, h + q:n] = bstage_ref[:, :].astype(jnp.bfloat16)

        send_ref[3] = mm(0, R)
        rd[3].start()

        fa.wait()
        a16_ref[1] = astage_ref[:, :].astype(jnp.bfloat16)
        fa = fetch_a(left)
        fa.start()

        send_ref[1] = mm(1, R)
        rd[1].start()
        send_ref[2] = mm(1, L)

        fa.wait()
        a16_ref[0] = astage_ref[:, :].astype(jnp.bfloat16)
        fa = fetch_a(my)
        fa.start()

        send_ref[4] = mm(0, L)
        rd[4].start()
        send_ref[5] = mm(0, R)

        rd[0].wait_recv()
        send_ref[2] = send_ref[2] + recv_ref[0]
        rd[2].start()
        rd[3].wait_recv()
        send_ref[5] = send_ref[5] + recv_ref[3]
        rd[5].start()

        fa.wait()
        a16_ref[1] = astage_ref[:, :].astype(jnp.bfloat16)
        out_ref[:, L] = mm(1, L)
        out_ref[:, R] = mm(1, R)

        rd[4].wait_recv()
        rd[2].wait_recv()
        out_ref[:, L] = out_ref[:, L] + (recv_ref[2] + recv_ref[4])
        rd[1].wait_recv()
        rd[5].wait_recv()
        out_ref[:, R] = out_ref[:, R] + (recv_ref[5] + recv_ref[1])

        for s in range(6):
            rd[s].wait_send()

    return pl.pallas_call(
        body,
        out_shape=jax.ShapeDtypeStruct((ch, n), jnp.bfloat16),
        in_specs=[
            pl.BlockSpec(memory_space=pltpu.MemorySpace.HBM),
            pl.BlockSpec(memory_space=pltpu.MemorySpace.HBM),
        ],
        out_specs=pl.BlockSpec(memory_space=pltpu.VMEM),
        scratch_shapes=[
            pltpu.VMEM((6, ch, h), jnp.bfloat16),
            pltpu.VMEM((6, ch, h), jnp.bfloat16),
            pltpu.VMEM((ch, k), jnp.float32),
            pltpu.VMEM((2, ch, k), jnp.bfloat16),
            pltpu.VMEM((k, q), jnp.float32),
            pltpu.VMEM((k, n), jnp.bfloat16),
            pltpu.SemaphoreType.DMA((6,)),
            pltpu.SemaphoreType.DMA((6,)),
            pltpu.SemaphoreType.DMA,
            pltpu.SemaphoreType.DMA,
        ],
        compiler_params=pltpu.CompilerParams(
            vmem_limit_bytes=110 * 1024 * 1024,
        ),
    )(A, B)


# device time: 99677 ns/iter; 1.5901x vs baseline; 1.1208x over previous
import jax
import jax.numpy as jnp
from jax import lax
from jax.experimental import pallas as pl
from jax.experimental.pallas import tpu as pltpu

N_DEV = 4


def kernel(A, B):
    m, k = A.shape
    _, n = B.shape
    ch = m // N_DEV
    h = n // 2
    q = n // 4

    def body(
        a_hbm,
        b_hbm,
        out_ref,
        send_ref,
        recv_ref,
        astage_ref,
        a16_ref,
        bstage_ref,
        b16_ref,
        send_sems,
        recv_sems,
        adma_sem,
        bdma_sem,
    ):
        my = lax.axis_index("i")
        right = (my + 1) % N_DEV
        left = (my + 3) % N_DEV
        diag = (my + 2) % N_DEV

        barrier = pltpu.get_barrier_semaphore()
        for nbr in (left, right):
            pl.semaphore_signal(
                barrier, inc=1, device_id=(nbr,),
                device_id_type=pl.DeviceIdType.MESH,
            )
        pl.semaphore_wait(barrier, 2)

        def fetch_a(rows):
            return pltpu.make_async_copy(
                a_hbm.at[pl.ds(rows * ch, ch), :], astage_ref, adma_sem
            )

        def fetch_b(j):
            return pltpu.make_async_copy(
                b_hbm.at[:, pl.ds(j * q, q)], bstage_ref, bdma_sem
            )

        def cast_b(j):
            b16_ref[:, pl.ds(j * q, q)] = bstage_ref[:, :].astype(jnp.bfloat16)

        def mm(slot, cols):
            return jnp.dot(
                a16_ref[slot, :, :],
                b16_ref[:, cols],
                preferred_element_type=jnp.float32,
            ).astype(jnp.bfloat16)

        def make(slot, tgt, sem, cols=None):
            src = send_ref.at[slot] if cols is None else send_ref.at[slot, :, cols]
            dst = recv_ref.at[slot] if cols is None else recv_ref.at[slot, :, cols]
            return pltpu.make_async_remote_copy(
                src_ref=src,
                dst_ref=dst,
                send_sem=send_sems.at[sem],
                recv_sem=recv_sems.at[sem],
                device_id=(tgt,),
                device_id_type=pl.DeviceIdType.MESH,
            )

        L = slice(0, h)
        R = slice(h, n)
        qa = pl.ds(0, q)
        qb = pl.ds(q, q)

        rd0a = make(0, right, 0, qa)
        rd0b = make(0, right, 6, qb)
        rd3a = make(3, left, 3, qa)
        rd3b = make(3, left, 7, qb)
        rd1 = make(1, right, 1)
        rd2 = make(2, right, 2)
        rd4 = make(4, left, 4)
        rd5 = make(5, left, 5)

        fa = fetch_a(diag)
        fa.start()
        fb = fetch_b(0)
        fb.start()
        fa.wait()
        a16_ref[0] = astage_ref[:, :].astype(jnp.bfloat16)
        fb.wait()
        cast_b(0)
        fb = fetch_b(2)
        fb.start()

        send_ref[0, :, qa] = mm(0, pl.ds(0, q))
        rd0a.start()
        fb.wait()
        cast_b(2)
        fb = fetch_b(1)
        fb.start()
        send_ref[3, :, qa] = mm(0, pl.ds(h, q))
        rd3a.start()
        fb.wait()
        cast_b(1)
        fb = fetch_b(3)
        fb.start()
        fa = fetch_a(right)
        fa.start()
        send_ref[0, :, qb] = mm(0, pl.ds(q, q))
        rd0b.start()
        fb.wait()
        cast_b(3)
        send_ref[3, :, qb] = mm(0, pl.ds(h + q, q))
        rd3b.start()

        fa.wait()
        a16_ref[1] = astage_ref[:, :].astype(jnp.bfloat16)
        fa = fetch_a(left)
        fa.start()
        send_ref[1] = mm(1, R)
        rd1.start()
        send_ref[2] = mm(1, L)

        fa.wait()
        a16_ref[0] = astage_ref[:, :].astype(jnp.bfloat16)
        fa = fetch_a(my)
        fa.start()
        send_ref[4] = mm(0, L)
        rd4.start()
        send_ref[5] = mm(0, R)

        rd0a.wait_recv()
        rd0b.wait_recv()
        send_ref[2] = send_ref[2] + recv_ref[0]
        rd2.start()
        rd3a.wait_recv()
        rd3b.wait_recv()
        send_ref[5] = send_ref[5] + recv_ref[3]
        rd5.start()

        fa.wait()
        a16_ref[1] = astage_ref[:, :].astype(jnp.bfloat16)
        out_ref[:, L] = mm(1, L)
        out_ref[:, R] = mm(1, R)

        rd4.wait_recv()
        rd2.wait_recv()
        out_ref[:, L] = out_ref[:, L] + (recv_ref[2] + recv_ref[4])
        rd1.wait_recv()
        rd5.wait_recv()
        out_ref[:, R] = out_ref[:, R] + (recv_ref[5] + recv_ref[1])

        for r in (rd0a, rd0b, rd3a, rd3b, rd1, rd2, rd4, rd5):
            r.wait_send()

    return pl.pallas_call(
        body,
        out_shape=jax.ShapeDtypeStruct((ch, n), jnp.bfloat16),
        in_specs=[
            pl.BlockSpec(memory_space=pltpu.MemorySpace.HBM),
            pl.BlockSpec(memory_space=pltpu.MemorySpace.HBM),
        ],
        out_specs=pl.BlockSpec(memory_space=pltpu.VMEM),
        scratch_shapes=[
            pltpu.VMEM((6, ch, h), jnp.bfloat16),
            pltpu.VMEM((6, ch, h), jnp.bfloat16),
            pltpu.VMEM((ch, k), jnp.float32),
            pltpu.VMEM((2, ch, k), jnp.bfloat16),
            pltpu.VMEM((k, q), jnp.float32),
            pltpu.VMEM((k, n), jnp.bfloat16),
            pltpu.SemaphoreType.DMA((8,)),
            pltpu.SemaphoreType.DMA((8,)),
            pltpu.SemaphoreType.DMA,
            pltpu.SemaphoreType.DMA,
        ],
        compiler_params=pltpu.CompilerParams(
            vmem_limit_bytes=110 * 1024 * 1024,
            collective_id=0,
        ),
    )(A, B)
